# Initial kernel scaffold; baseline (speedup 1.0000x reference)
import jax
import jax.numpy as jnp
from jax.experimental import pallas as pl


def kernel(R, Z, neighbor_idx, box, W1, b1, W2, b2, scale, shift):
    raise NotImplementedError("write your pallas kernel here")


# speedup vs baseline: 12.9849x; 12.9849x over previous
import functools

import jax
import jax.numpy as jnp
from jax import lax
from jax.experimental import pallas as pl
from jax.experimental.pallas import tpu as pltpu
from jax.experimental.pallas import tpu_sc as plsc

_N = 50000
_K = 32
_NB = 16
_HID = 64
_NSPEC = 10
_CUTOFF = 5.0
_GAMMA = (_NB / _CUTOFF) ** 2

_NC = 2
_NS = 16
_L = 16
_NW = _NC * _NS

_NPAD = 50176
_TOT = _K * _NPAD
_PER_W = _TOT // _NW
_CHUNK = 6272


def _sc_gather(tables, idx_flat):
    mesh = plsc.VectorSubcoreMesh(core_axis_name="c", subcore_axis_name="s")

    @functools.partial(
        pl.kernel,
        out_type=jax.ShapeDtypeStruct((3 * _TOT,), jnp.float32),
        mesh=mesh,
        scratch_types=[
            pltpu.VMEM((_N,), jnp.float32),
            pltpu.VMEM((_CHUNK,), jnp.int32),
            pltpu.VMEM((_CHUNK,), jnp.float32),
        ],
        compiler_params=pltpu.CompilerParams(needs_layout_passes=False),
    )
    def gather_kernel(tab_hbm, idx_hbm, out_hbm, tab_v, idx_v, val_v):
        wid = lax.axis_index("s") * _NC + lax.axis_index("c")
        base = wid * _PER_W
        for comp in range(3):
            pltpu.sync_copy(tab_hbm.at[pl.ds(comp * _N, _N)], tab_v)
            for cb in range(_PER_W // _CHUNK):
                off = base + cb * _CHUNK
                pltpu.sync_copy(idx_hbm.at[pl.ds(off, _CHUNK)], idx_v)

                @pl.loop(0, _CHUNK // _L, unroll=8)
                def _(j):
                    sl = pl.ds(j * _L, _L)
                    val_v[sl] = plsc.load_gather(tab_v, [idx_v[sl]])

                pltpu.sync_copy(
                    val_v, out_hbm.at[pl.ds(comp * _TOT + off, _CHUNK)])

    return gather_kernel(tables, idx_flat)


def _tc_body(rjx, rjy, rjz, idxt, rx, ry, rz, zt, w1t, b1, w2t,
             b2_s, scale_s, shift_s, out_ref):
    i = pl.program_id(0)
    bn = rjx.shape[1]
    f32 = jnp.float32

    dx = rjx[...] - rx[...]
    dy = rjy[...] - ry[...]
    dz = rjz[...] - rz[...]
    d2 = dx * dx + dy * dy + dz * dz + 1e-12
    d = jnp.sqrt(d2)

    atom = i * bn + lax.broadcasted_iota(jnp.int32, (_K, bn), 1)
    idxv = idxt[...]
    valid = (idxv < _N) & (idxv != atom)
    fc = 0.5 * (jnp.cos(jnp.pi / _CUTOFF * d) + 1.0) * (d < _CUTOFF).astype(f32)
    w = fc * valid.astype(f32)

    mu = (lax.broadcasted_iota(jnp.int32, (_NB, 1, 1), 0).astype(f32)
          * (_CUTOFF / (_NB - 1)))
    t = d[None] - mu
    basis = jnp.exp(-_GAMMA * (t * t)) * w[None]

    m0 = jnp.sum(basis, axis=1)
    inv_d = 1.0 / (d + 1e-9)
    m1x = jnp.sum(basis * (dx * inv_d)[None], axis=1)
    m1y = jnp.sum(basis * (dy * inv_d)[None], axis=1)
    m1z = jnp.sum(basis * (dz * inv_d)[None], axis=1)
    inv1 = m1x * m1x + m1y * m1y + m1z * m1z

    gm = jnp.concatenate([m0, inv1], axis=0)
    h = jnp.dot(w1t[...], gm, preferred_element_type=f32) + b1[...]
    h = h * (1.0 / (1.0 + jnp.exp(-h)))
    o = jnp.dot(w2t[...], h, preferred_element_type=f32) + b2_s[0]

    zb = zt[...]
    sc = jnp.zeros((1, bn), f32)
    sh = jnp.zeros((1, bn), f32)
    for s in range(_NSPEC):
        sc = jnp.where(zb == s, scale_s[s], sc)
        sh = jnp.where(zb == s, shift_s[s], sh)
    atomic = o * sc + sh
    atomic = jnp.where(zb > 0, atomic, 0.0)
    part = jnp.sum(atomic)

    @pl.when(i == 0)
    def _():
        out_ref[0, 0] = part

    @pl.when(i > 0)
    def _():
        out_ref[0, 0] += part


def _tc_energy(rjx, rjy, rjz, idxt, rx, ry, rz, zt, w1t, b1, w2t,
               b2, scale, shift, bn=512):
    grid = (_NPAD // bn,)
    edge = lambda i: (0, i)
    const = lambda i: (0, 0)
    return pl.pallas_call(
        _tc_body,
        grid=grid,
        in_specs=[
            pl.BlockSpec((_K, bn), edge),
            pl.BlockSpec((_K, bn), edge),
            pl.BlockSpec((_K, bn), edge),
            pl.BlockSpec((_K, bn), edge),
            pl.BlockSpec((1, bn), edge),
            pl.BlockSpec((1, bn), edge),
            pl.BlockSpec((1, bn), edge),
            pl.BlockSpec((1, bn), edge),
            pl.BlockSpec((_HID, 2 * _NB), const),
            pl.BlockSpec((_HID, 1), const),
            pl.BlockSpec((1, _HID), const),
            pl.BlockSpec(memory_space=pltpu.SMEM),
            pl.BlockSpec(memory_space=pltpu.SMEM),
            pl.BlockSpec(memory_space=pltpu.SMEM),
        ],
        out_specs=pl.BlockSpec(memory_space=pltpu.SMEM),
        out_shape=jax.ShapeDtypeStruct((1, 1), jnp.float32),
        compiler_params=pltpu.CompilerParams(
            dimension_semantics=("arbitrary",)),
    )(rjx, rjy, rjz, idxt, rx, ry, rz, zt, w1t, b1, w2t, b2, scale, shift)


def kernel(R, Z, neighbor_idx, box, W1, b1, W2, b2, scale, shift):
    del box
    pad = _NPAD - _N
    tables = R.T.reshape(-1)
    idxt = jnp.pad(neighbor_idx, ((0, pad), (0, 0))).T
    gathered = _sc_gather(tables, idxt.reshape(-1))
    rj = gathered.reshape(3, _K, _NPAD)
    rpt = jnp.pad(R, ((0, pad), (0, 0))).T
    zt = jnp.pad(Z, (0, pad)).reshape(1, _NPAD)
    out = _tc_energy(
        rj[0], rj[1], rj[2], idxt,
        rpt[0:1], rpt[1:2], rpt[2:3], zt,
        W1.T, b1.reshape(_HID, 1), W2.T,
        b2, scale, shift)
    return jnp.sum(out, dtype=jnp.float64)


# speedup vs baseline: 15.6316x; 1.2038x over previous
import functools

import jax
import jax.numpy as jnp
from jax import lax
from jax.experimental import pallas as pl
from jax.experimental.pallas import tpu as pltpu
from jax.experimental.pallas import tpu_sc as plsc

_N = 50000
_K = 32
_NB = 16
_HID = 64
_NSPEC = 10
_CUTOFF = 5.0
_GAMMA = (_NB / _CUTOFF) ** 2

_NC = 2
_NS = 16
_L = 16
_NW = _NC * _NS

_NPAD = 50176
_TOT = _K * _NPAD
_PER_W = _TOT // _NW
_CHUNK = 6272


def _sc_gather(tables, idx_flat):
    mesh = plsc.VectorSubcoreMesh(core_axis_name="c", subcore_axis_name="s")

    @functools.partial(
        pl.kernel,
        out_type=jax.ShapeDtypeStruct((3 * _TOT,), jnp.float32),
        mesh=mesh,
        scratch_types=[
            pltpu.VMEM((_N,), jnp.float32),
            pltpu.VMEM((_CHUNK,), jnp.int32),
            pltpu.VMEM((_CHUNK,), jnp.float32),
        ],
        compiler_params=pltpu.CompilerParams(needs_layout_passes=False),
    )
    def gather_kernel(tab_hbm, idx_hbm, out_hbm, tab_v, idx_v, val_v):
        wid = lax.axis_index("s") * _NC + lax.axis_index("c")
        base = wid * _PER_W
        for comp in range(3):
            pltpu.sync_copy(tab_hbm.at[pl.ds(comp * _N, _N)], tab_v)
            for cb in range(_PER_W // _CHUNK):
                off = base + cb * _CHUNK
                pltpu.sync_copy(idx_hbm.at[pl.ds(off, _CHUNK)], idx_v)

                @pl.loop(0, _CHUNK // _L, unroll=8)
                def _(j):
                    sl = pl.ds(j * _L, _L)
                    val_v[sl] = plsc.load_gather(tab_v, [idx_v[sl]])

                pltpu.sync_copy(
                    val_v, out_hbm.at[pl.ds(comp * _TOT + off, _CHUNK)])

    return gather_kernel(tables, idx_flat)


def _tc_body(rjx, rjy, rjz, idxt, rx, ry, rz, gm_ref):
    i = pl.program_id(0)
    bn = rjx.shape[1]
    f32 = jnp.float32

    dx = rjx[...] - rx[...]
    dy = rjy[...] - ry[...]
    dz = rjz[...] - rz[...]
    d2 = dx * dx + dy * dy + dz * dz + 1e-12
    d = jnp.sqrt(d2)

    atom = i * bn + lax.broadcasted_iota(jnp.int32, (_K, bn), 1)
    idxv = idxt[...]
    valid = (idxv < _N) & (idxv != atom)
    fc = 0.5 * (jnp.cos(jnp.pi / _CUTOFF * d) + 1.0) * (d < _CUTOFF).astype(f32)
    w = fc * valid.astype(f32)

    mu_sub = (lax.broadcasted_iota(jnp.int32, (_NB, bn), 0).astype(f32)
              * (_CUTOFF / (_NB - 1)))
    inv_d = 1.0 / (d + 1e-9)
    ux = dx * inv_d
    uy = dy * inv_d
    uz = dz * inv_d

    m0 = jnp.zeros((_NB, bn), f32)
    m1x = jnp.zeros((_NB, bn), f32)
    m1y = jnp.zeros((_NB, bn), f32)
    m1z = jnp.zeros((_NB, bn), f32)
    neg_gamma_log2e = -_GAMMA * 1.4426950408889634
    for k in range(_K):
        s = jnp.broadcast_to(d[k:k + 1, :], (_NB, bn))
        t = s - mu_sub
        e = jnp.exp2((t * t) * neg_gamma_log2e)
        e = e * jnp.broadcast_to(w[k:k + 1, :], (_NB, bn))
        m0 = m0 + e
        m1x = m1x + e * jnp.broadcast_to(ux[k:k + 1, :], (_NB, bn))
        m1y = m1y + e * jnp.broadcast_to(uy[k:k + 1, :], (_NB, bn))
        m1z = m1z + e * jnp.broadcast_to(uz[k:k + 1, :], (_NB, bn))
    inv1 = m1x * m1x + m1y * m1y + m1z * m1z
    gm_ref[...] = jnp.concatenate([m0, inv1], axis=0)


def _mlp_body(gm, zt, w1t, b1, w2t, b2_s, scale_s, shift_s, out_ref, acc):
    i = pl.program_id(0)
    nsteps = pl.num_programs(0)
    bn = gm.shape[1]
    f32 = jnp.float32

    h = jnp.dot(w1t[...], gm[...], preferred_element_type=f32) + b1[...]
    h = h * (1.0 / (1.0 + jnp.exp(-h)))
    o = jnp.dot(w2t[...], h, preferred_element_type=f32) + b2_s[0]

    zb = zt[...]
    sc = jnp.zeros((1, bn), f32)
    sh = jnp.zeros((1, bn), f32)
    for s in range(_NSPEC):
        sc = jnp.where(zb == s, scale_s[s], sc)
        sh = jnp.where(zb == s, shift_s[s], sh)
    atomic = o * sc + sh
    atomic = jnp.where(zb > 0, atomic, 0.0)

    @pl.when(i == 0)
    def _():
        acc[...] = atomic

    @pl.when(i > 0)
    def _():
        acc[...] += atomic

    @pl.when(i == nsteps - 1)
    def _():
        out_ref[0, 0] = jnp.sum(acc[...])


def _tc_energy(rjx, rjy, rjz, idxt, rx, ry, rz, zt, w1t, b1, w2t,
               b2, scale, shift, bn=512, bn2=3584):
    edge = lambda i: (0, i)
    const = lambda i: (0, 0)
    gm = pl.pallas_call(
        _tc_body,
        grid=(_NPAD // bn,),
        in_specs=[
            pl.BlockSpec((_K, bn), edge),
            pl.BlockSpec((_K, bn), edge),
            pl.BlockSpec((_K, bn), edge),
            pl.BlockSpec((_K, bn), edge),
            pl.BlockSpec((1, bn), edge),
            pl.BlockSpec((1, bn), edge),
            pl.BlockSpec((1, bn), edge),
        ],
        out_specs=pl.BlockSpec((2 * _NB, bn), edge),
        out_shape=jax.ShapeDtypeStruct((2 * _NB, _NPAD), jnp.float32),
        compiler_params=pltpu.CompilerParams(
            dimension_semantics=("arbitrary",)),
    )(rjx, rjy, rjz, idxt, rx, ry, rz)
    return pl.pallas_call(
        _mlp_body,
        grid=(_NPAD // bn2,),
        in_specs=[
            pl.BlockSpec((2 * _NB, bn2), edge),
            pl.BlockSpec((1, bn2), edge),
            pl.BlockSpec((_HID, 2 * _NB), const),
            pl.BlockSpec((_HID, 1), const),
            pl.BlockSpec((1, _HID), const),
            pl.BlockSpec(memory_space=pltpu.SMEM),
            pl.BlockSpec(memory_space=pltpu.SMEM),
            pl.BlockSpec(memory_space=pltpu.SMEM),
        ],
        out_specs=pl.BlockSpec(memory_space=pltpu.SMEM),
        out_shape=jax.ShapeDtypeStruct((1, 1), jnp.float32),
        scratch_shapes=[pltpu.VMEM((1, bn2), jnp.float32)],
        compiler_params=pltpu.CompilerParams(
            dimension_semantics=("arbitrary",)),
    )(gm, zt, w1t, b1, w2t, b2, scale, shift)


def kernel(R, Z, neighbor_idx, box, W1, b1, W2, b2, scale, shift):
    del box
    pad = _NPAD - _N
    tables = R.T.reshape(-1)
    idxt = jnp.pad(neighbor_idx, ((0, pad), (0, 0))).T
    gathered = _sc_gather(tables, idxt.reshape(-1))
    rj = gathered.reshape(3, _K, _NPAD)
    rpt = jnp.pad(R, ((0, pad), (0, 0))).T
    zt = jnp.pad(Z, (0, pad)).reshape(1, _NPAD)
    out = _tc_energy(
        rj[0], rj[1], rj[2], idxt,
        rpt[0:1], rpt[1:2], rpt[2:3], zt,
        W1.T, b1.reshape(_HID, 1), W2.T,
        b2, scale, shift)
    return jnp.sum(out, dtype=jnp.float64)


# speedup vs baseline: 16.5979x; 1.0618x over previous
import functools

import jax
import jax.numpy as jnp
from jax import lax
from jax.experimental import pallas as pl
from jax.experimental.pallas import tpu as pltpu
from jax.experimental.pallas import tpu_sc as plsc

_N = 50000
_K = 32
_NB = 16
_HID = 64
_NSPEC = 10
_CUTOFF = 5.0
_GAMMA = (_NB / _CUTOFF) ** 2

_NC = 2
_NS = 16
_L = 16
_NW = _NC * _NS

_NPAD = 50176
_TOT = _K * _NPAD
_PER_W = _TOT // _NW
_CHUNK = 6272


def _sc_gather(tables, idx_flat):
    mesh = plsc.VectorSubcoreMesh(core_axis_name="c", subcore_axis_name="s")

    @functools.partial(
        pl.kernel,
        out_type=[jax.ShapeDtypeStruct((_TOT,), jnp.float32)] * 3,
        mesh=mesh,
        scratch_types=[
            pltpu.VMEM((_N,), jnp.float32),
            pltpu.VMEM((_CHUNK,), jnp.int32),
            pltpu.VMEM((_CHUNK,), jnp.float32),
        ],
        compiler_params=pltpu.CompilerParams(needs_layout_passes=False),
    )
    def gather_kernel(tab_hbm, idx_hbm, out_x, out_y, out_z, tab_v, idx_v, val_v):
        wid = lax.axis_index("s") * _NC + lax.axis_index("c")
        base = wid * _PER_W
        outs = [out_x, out_y, out_z]
        for comp in range(3):
            pltpu.sync_copy(tab_hbm.at[pl.ds(comp * _N, _N)], tab_v)
            for cb in range(_PER_W // _CHUNK):
                off = base + cb * _CHUNK
                pltpu.sync_copy(idx_hbm.at[pl.ds(off, _CHUNK)], idx_v)

                @pl.loop(0, _CHUNK // _L, unroll=8)
                def _(j):
                    sl = pl.ds(j * _L, _L)
                    val_v[sl] = plsc.load_gather(tab_v, [idx_v[sl]])

                pltpu.sync_copy(val_v, outs[comp].at[pl.ds(off, _CHUNK)])

    return gather_kernel(tables, idx_flat)


def _tc_body(rjx, rjy, rjz, idxt, rx, ry, rz, gm_ref):
    i = pl.program_id(0)
    bn = rjx.shape[1]
    f32 = jnp.float32

    dx = rjx[...] - rx[...]
    dy = rjy[...] - ry[...]
    dz = rjz[...] - rz[...]
    d2 = dx * dx + dy * dy + dz * dz + 1e-12
    d = jnp.sqrt(d2)

    atom = i * bn + lax.broadcasted_iota(jnp.int32, (_K, bn), 1)
    idxv = idxt[...]
    valid = (idxv < _N) & (idxv != atom)
    fc = 0.5 * (jnp.cos(jnp.pi / _CUTOFF * d) + 1.0) * (d < _CUTOFF).astype(f32)
    w = fc * valid.astype(f32)

    mu_sub = (lax.broadcasted_iota(jnp.int32, (_NB, bn), 0).astype(f32)
              * (_CUTOFF / (_NB - 1)))
    inv_d = 1.0 / (d + 1e-9)
    ux = dx * inv_d
    uy = dy * inv_d
    uz = dz * inv_d

    m0 = jnp.zeros((_NB, bn), f32)
    m1x = jnp.zeros((_NB, bn), f32)
    m1y = jnp.zeros((_NB, bn), f32)
    m1z = jnp.zeros((_NB, bn), f32)
    neg_gamma_log2e = -_GAMMA * 1.4426950408889634
    for k in range(_K):
        s = jnp.broadcast_to(d[k:k + 1, :], (_NB, bn))
        t = s - mu_sub
        e = jnp.exp2((t * t) * neg_gamma_log2e)
        e = e * jnp.broadcast_to(w[k:k + 1, :], (_NB, bn))
        m0 = m0 + e
        m1x = m1x + e * jnp.broadcast_to(ux[k:k + 1, :], (_NB, bn))
        m1y = m1y + e * jnp.broadcast_to(uy[k:k + 1, :], (_NB, bn))
        m1z = m1z + e * jnp.broadcast_to(uz[k:k + 1, :], (_NB, bn))
    inv1 = m1x * m1x + m1y * m1y + m1z * m1z
    gm_ref[...] = jnp.concatenate([m0, inv1], axis=0)


def _mlp_body(gm, zt, w1t, b1, w2t, b2_s, scale_s, shift_s, out_ref, acc):
    i = pl.program_id(0)
    nsteps = pl.num_programs(0)
    bn = gm.shape[1]
    f32 = jnp.float32

    h = jnp.dot(w1t[...], gm[...], preferred_element_type=f32) + b1[...]
    h = h * (1.0 / (1.0 + jnp.exp(-h)))
    o = jnp.dot(w2t[...], h, preferred_element_type=f32) + b2_s[0]

    zb = zt[...]
    sc = jnp.zeros((1, bn), f32)
    sh = jnp.zeros((1, bn), f32)
    for s in range(_NSPEC):
        sc = jnp.where(zb == s, scale_s[s], sc)
        sh = jnp.where(zb == s, shift_s[s], sh)
    atomic = o * sc + sh
    atomic = jnp.where(zb > 0, atomic, 0.0)

    @pl.when(i == 0)
    def _():
        acc[...] = atomic

    @pl.when(i > 0)
    def _():
        acc[...] += atomic

    @pl.when(i == nsteps - 1)
    def _():
        out_ref[0, 0] = jnp.sum(acc[...])


def _tc_energy(rjx, rjy, rjz, idxt, rx, ry, rz, zt, w1t, b1, w2t,
               b2, scale, shift, bn=512, bn2=3584):
    edge = lambda i: (0, i)
    const = lambda i: (0, 0)
    gm = pl.pallas_call(
        _tc_body,
        grid=(_NPAD // bn,),
        in_specs=[
            pl.BlockSpec((_K, bn), edge),
            pl.BlockSpec((_K, bn), edge),
            pl.BlockSpec((_K, bn), edge),
            pl.BlockSpec((_K, bn), edge),
            pl.BlockSpec((1, bn), edge),
            pl.BlockSpec((1, bn), edge),
            pl.BlockSpec((1, bn), edge),
        ],
        out_specs=pl.BlockSpec((2 * _NB, bn), edge),
        out_shape=jax.ShapeDtypeStruct((2 * _NB, _NPAD), jnp.float32),
        compiler_params=pltpu.CompilerParams(
            dimension_semantics=("arbitrary",)),
    )(rjx, rjy, rjz, idxt, rx, ry, rz)
    return pl.pallas_call(
        _mlp_body,
        grid=(_NPAD // bn2,),
        in_specs=[
            pl.BlockSpec((2 * _NB, bn2), edge),
            pl.BlockSpec((1, bn2), edge),
            pl.BlockSpec((_HID, 2 * _NB), const),
            pl.BlockSpec((_HID, 1), const),
            pl.BlockSpec((1, _HID), const),
            pl.BlockSpec(memory_space=pltpu.SMEM),
            pl.BlockSpec(memory_space=pltpu.SMEM),
            pl.BlockSpec(memory_space=pltpu.SMEM),
        ],
        out_specs=pl.BlockSpec(memory_space=pltpu.SMEM),
        out_shape=jax.ShapeDtypeStruct((1, 1), jnp.float32),
        scratch_shapes=[pltpu.VMEM((1, bn2), jnp.float32)],
        compiler_params=pltpu.CompilerParams(
            dimension_semantics=("arbitrary",)),
    )(gm, zt, w1t, b1, w2t, b2, scale, shift)


def kernel(R, Z, neighbor_idx, box, W1, b1, W2, b2, scale, shift):
    del box
    pad = _NPAD - _N
    tables = R.T.reshape(-1)
    idxt = jnp.pad(neighbor_idx, ((0, pad), (0, 0))).T
    gx, gy, gz = _sc_gather(tables, idxt.reshape(-1))
    rjx = gx.reshape(_K, _NPAD)
    rjy = gy.reshape(_K, _NPAD)
    rjz = gz.reshape(_K, _NPAD)
    rpt = jnp.pad(R, ((0, pad), (0, 0))).T
    zt = jnp.pad(Z, (0, pad)).reshape(1, _NPAD)
    out = _tc_energy(
        rjx, rjy, rjz, idxt,
        rpt[0:1], rpt[1:2], rpt[2:3], zt,
        W1.T, b1.reshape(_HID, 1), W2.T,
        b2, scale, shift)
    return jnp.sum(out, dtype=jnp.float64)
